# baseline (speedup 1.0000x reference)
import dataclasses
import functools

import jax
import jax.numpy as jnp
from jax import lax
from jax.experimental import pallas as pl
from jax.experimental.pallas import tpu as pltpu
from jax.experimental.pallas import tpu_sc as plsc

NC, NS = 2, 16
NW = NC * NS
CHUNK = 88
RING = 4


def _sc_mesh():
    return plsc.VectorSubcoreMesh(
        core_axis_name="c", subcore_axis_name="s", num_cores=NC, num_subcores=NS
    )


def _sc_params():
    cp = pltpu.CompilerParams()
    if "needs_layout_passes" in pltpu.CompilerParams.__dataclass_fields__:
        cp = dataclasses.replace(cp, needs_layout_passes=False)
    return cp


def _hist_sc(n_pad, per_w):
    assert per_w % 16 == 0 and n_pad % 16 == 0

    @functools.partial(
        pl.kernel,
        out_type=jax.ShapeDtypeStruct((NC, NS, n_pad), jnp.float32),
        mesh=_sc_mesh(),
        compiler_params=_sc_params(),
        scratch_types=[
            pltpu.VMEM((n_pad,), jnp.float32),
            pltpu.VMEM((per_w,), jnp.int32),
            pltpu.SemaphoreType.DMA,
        ],
    )
    def hist(dst_hbm, out_hbm, hist_v, idx_v, isem):
        c = lax.axis_index("c")
        s = lax.axis_index("s")
        w = c * NS + s
        pltpu.async_copy(dst_hbm.at[pl.ds(w * per_w, per_w)], idx_v, isem)
        zero16 = jnp.zeros((16,), jnp.float32)

        @pl.loop(0, n_pad, step=16)
        def _(i):
            hist_v[pl.ds(i, 16)] = zero16

        pltpu.make_async_copy(dst_hbm.at[pl.ds(0, per_w)], idx_v, isem).wait()
        ones16 = jnp.ones((16,), jnp.float32)

        @pl.loop(0, per_w, step=16)
        def _(k):
            plsc.addupdate_scatter(hist_v, [idx_v[pl.ds(k, 16)]], ones16)

        pltpu.sync_copy(hist_v, out_hbm.at[c].at[s])

    return hist


def _edge_accumulate_sc(n_pad, d, nchunks):
    rows_per_sub = n_pad // NS
    assert nchunks % RING == 0 and nchunks >= 2 * RING

    @functools.partial(
        pl.kernel,
        out_type=jax.ShapeDtypeStruct((NC, n_pad, d), jnp.float32),
        mesh=_sc_mesh(),
        scratch_types=(
            [pltpu.VMEM_SHARED((n_pad, d), jnp.float32)]
            + [pltpu.VMEM((RING, CHUNK), jnp.int32)] * 2
            + [pltpu.VMEM((CHUNK, d), jnp.float32)] * RING
            + [pltpu.SemaphoreType.DMA] * (3 * RING)
        ),
    )
    def accum(table_hbm, src_hbm, dst_hbm, zeros_hbm, out_hbm,
              acc, sidx_v, didx_v, *bufs_and_sems):
        rows = list(bufs_and_sems[:RING])
        isem = list(bufs_and_sems[RING:2 * RING])
        gsem = list(bufs_and_sems[2 * RING:3 * RING])
        ssem = list(bufs_and_sems[3 * RING:])
        c = lax.axis_index("c")
        s = lax.axis_index("s")
        w = c * NS + s
        row0 = s * rows_per_sub
        pltpu.sync_copy(zeros_hbm.at[pl.ds(row0, rows_per_sub), :],
                        acc.at[pl.ds(row0, rows_per_sub), :])
        plsc.subcore_barrier()

        def issue_idx(j, b):
            pltpu.async_copy(src_hbm.at[w, j], sidx_v.at[b], isem[b])
            pltpu.async_copy(dst_hbm.at[w, j], didx_v.at[b], isem[b])

        def wait_idx(b):
            pltpu.make_async_copy(src_hbm.at[w, 0], sidx_v.at[b],
                                  isem[b]).wait()
            pltpu.make_async_copy(dst_hbm.at[w, 0], didx_v.at[b],
                                  isem[b]).wait()

        def issue_gather(b):
            pltpu.async_copy(table_hbm.at[sidx_v.at[b]], rows[b], gsem[b])

        def wait_gather(b):
            pltpu.make_async_copy(table_hbm.at[sidx_v.at[0]], rows[b],
                                  gsem[b]).wait()

        def issue_scatter(b):
            pltpu.async_copy(rows[b], acc.at[didx_v.at[b]], ssem[b], add=True)

        def wait_scatter(b):
            pltpu.make_async_copy(rows[b], acc.at[didx_v.at[0]],
                                  ssem[b]).wait()

        def stage(i, with_idx=True, with_gather=True, with_scatter=True,
                  with_free=True):
            if with_idx:
                if with_free:
                    wait_scatter(i % RING)
                issue_idx(i, i % RING)
            if with_gather:
                wait_idx((i - 1) % RING)
                issue_gather((i - 1) % RING)
            if with_scatter:
                wait_gather((i - 2) % RING)
                issue_scatter((i - 2) % RING)

        for i in range(RING):
            stage(i, with_gather=i >= 1, with_scatter=i >= 2, with_free=False)

        @pl.loop(1, nchunks // RING)
        def _(jq):
            for k in range(RING):
                b = k
                wait_scatter(b)
                pltpu.async_copy(src_hbm.at[w, jq * RING + k], sidx_v.at[b],
                                 isem[b])
                pltpu.async_copy(dst_hbm.at[w, jq * RING + k], didx_v.at[b],
                                 isem[b])
                wait_idx((k - 1) % RING)
                issue_gather((k - 1) % RING)
                wait_gather((k - 2) % RING)
                issue_scatter((k - 2) % RING)

        stage(nchunks, with_idx=False)
        stage(nchunks + 1, with_idx=False, with_gather=False)
        for b in range(RING):
            wait_scatter(b)
        plsc.subcore_barrier()
        pltpu.sync_copy(acc.at[pl.ds(row0, rows_per_sub), :],
                        out_hbm.at[c].at[pl.ds(row0, rows_per_sub), :])

    return accum


def _tc_matmul(x, w, rb):
    n, k = x.shape
    _, m = w.shape

    def body(x_ref, w_ref, o_ref):
        o_ref[...] = jnp.dot(x_ref[...], w_ref[...],
                             preferred_element_type=jnp.float32)

    return pl.pallas_call(
        body,
        grid=(n // rb,),
        in_specs=[pl.BlockSpec((rb, k), lambda i: (i, 0)),
                  pl.BlockSpec((k, m), lambda i: (0, 0))],
        out_specs=pl.BlockSpec((rb, m), lambda i: (i, 0)),
        out_shape=jax.ShapeDtypeStruct((n, m), jnp.float32),
    )(x, w)


def _tc_dis_prescale(hist2, h1, rb):
    n, d = h1.shape

    def body(h_ref, h1_ref, dis_ref, g_ref):
        deg = jnp.sum(h_ref[...], axis=0) + 1.0
        dis_c = lax.rsqrt(deg).reshape(rb, 1)
        dis_ref[...] = dis_c
        g_ref[...] = h1_ref[...] * dis_c

    return pl.pallas_call(
        body,
        grid=(n // rb,),
        in_specs=[pl.BlockSpec((NW, rb), lambda i: (0, i)),
                  pl.BlockSpec((rb, d), lambda i: (i, 0))],
        out_specs=[pl.BlockSpec((rb, 1), lambda i: (i, 0)),
                   pl.BlockSpec((rb, d), lambda i: (i, 0))],
        out_shape=[jax.ShapeDtypeStruct((n, 1), jnp.float32),
                   jax.ShapeDtypeStruct((n, d), jnp.float32)],
    )(hist2, h1)


def _tc_layer_mid(acc, h1, dis, b1, w2, rb):
    _, n, d = acc.shape
    m = w2.shape[1]

    def body(acc_ref, h1_ref, dis_ref, b1_ref, w2_ref, h2_ref, g2_ref):
        dis_c = dis_ref[...]
        su = acc_ref[0] + acc_ref[1]
        z = jnp.maximum(dis_c * su + (dis_c * dis_c) * h1_ref[...] + b1_ref[...],
                        0.0)
        h2 = jnp.dot(z, w2_ref[...], preferred_element_type=jnp.float32)
        h2_ref[...] = h2
        g2_ref[...] = h2 * dis_c

    return pl.pallas_call(
        body,
        grid=(n // rb,),
        in_specs=[pl.BlockSpec((NC, rb, d), lambda i: (0, i, 0)),
                  pl.BlockSpec((rb, d), lambda i: (i, 0)),
                  pl.BlockSpec((rb, 1), lambda i: (i, 0)),
                  pl.BlockSpec((1, d), lambda i: (0, 0)),
                  pl.BlockSpec((d, m), lambda i: (0, 0))],
        out_specs=[pl.BlockSpec((rb, m), lambda i: (i, 0)),
                   pl.BlockSpec((rb, m), lambda i: (i, 0))],
        out_shape=[jax.ShapeDtypeStruct((n, m), jnp.float32),
                   jax.ShapeDtypeStruct((n, m), jnp.float32)],
    )(acc, h1, dis, b1, w2)


def _tc_layer_out(acc, h2, dis, b2, n_out, rb):
    _, _, d = acc.shape

    def body(acc_ref, h2_ref, dis_ref, b2_ref, o_ref):
        dis_c = dis_ref[...]
        su = acc_ref[0] + acc_ref[1]
        o_ref[...] = dis_c * su + (dis_c * dis_c) * h2_ref[...] + b2_ref[...]

    return pl.pallas_call(
        body,
        grid=(n_out // rb,),
        in_specs=[pl.BlockSpec((NC, rb, d), lambda i: (0, i, 0)),
                  pl.BlockSpec((rb, d), lambda i: (i, 0)),
                  pl.BlockSpec((rb, 1), lambda i: (i, 0)),
                  pl.BlockSpec((1, d), lambda i: (0, 0))],
        out_specs=pl.BlockSpec((rb, d), lambda i: (i, 0)),
        out_shape=jax.ShapeDtypeStruct((n_out, d), jnp.float32),
    )(acc, h2, dis, b2)


def kernel(x, edge_index, W1, b1, W2, b2):
    n, _ = x.shape
    e = edge_index.shape[1]
    d_h = W1.shape[1]
    d_out = W2.shape[1]

    n_pad = ((n + 80 * NS - 1) // (80 * NS)) * (80 * NS)
    rb = n_pad // 10
    assert rb % 8 == 0

    nchunks = -(-e // (NW * CHUNK))
    nchunks = max(2 * RING, ((nchunks + RING - 1) // RING) * RING)
    e_pad = NW * CHUNK * nchunks
    if e_pad > e and n_pad == n:
        n_pad += 80 * NS
    pad_len = e_pad - e
    sink = n + (jnp.arange(pad_len, dtype=jnp.int32) % (n_pad - n)) \
        if pad_len else jnp.zeros((0,), jnp.int32)
    src = jnp.concatenate([edge_index[0], sink]).reshape(NW, nchunks, CHUNK)
    dst_flat = jnp.concatenate([edge_index[1], sink])
    dst = dst_flat.reshape(NW, nchunks, CHUNK)

    x_pad = jnp.pad(x, ((0, n_pad - n), (0, 0)))
    zeros_tab = jnp.zeros((n_pad, d_h), jnp.float32)

    hist = _hist_sc(n_pad, nchunks * CHUNK)(dst_flat)
    hist2 = hist.reshape(NW, n_pad)

    h1 = _tc_matmul(x_pad, W1, rb)
    dis, g1 = _tc_dis_prescale(hist2, h1, rb)

    acc1 = _edge_accumulate_sc(n_pad, d_h, nchunks)(g1, src, dst, zeros_tab)

    h2, g2 = _tc_layer_mid(acc1, h1, dis, b1.reshape(1, d_h), W2, rb)

    acc2 = _edge_accumulate_sc(n_pad, d_out, nchunks)(g2, src, dst, zeros_tab)

    rb_out = rb
    if n % rb:
        for div in (10, 8, 5, 4, 2, 1):
            if n % div == 0 and (n // div) % 8 == 0:
                rb_out = n // div
                break
    else:
        rb_out = rb if n % rb == 0 else n
    return _tc_layer_out(acc2, h2, dis, b2.reshape(1, d_out), n, rb_out)


# speedup vs baseline: 1.0447x; 1.0447x over previous
import dataclasses
import functools

import jax
import jax.numpy as jnp
from jax import lax
from jax.experimental import pallas as pl
from jax.experimental.pallas import tpu as pltpu
from jax.experimental.pallas import tpu_sc as plsc

NC, NS = 2, 16
NW = NC * NS
CHUNK = 88
RING = 4


def _sc_mesh():
    return plsc.VectorSubcoreMesh(
        core_axis_name="c", subcore_axis_name="s", num_cores=NC, num_subcores=NS
    )


def _sc_params():
    cp = pltpu.CompilerParams()
    if "needs_layout_passes" in pltpu.CompilerParams.__dataclass_fields__:
        cp = dataclasses.replace(cp, needs_layout_passes=False)
    return cp


def _hist_sc(n_pad, per_w):
    assert per_w % 16 == 0 and n_pad % 16 == 0

    @functools.partial(
        pl.kernel,
        out_type=jax.ShapeDtypeStruct((NC, NS, n_pad), jnp.float32),
        mesh=_sc_mesh(),
        compiler_params=_sc_params(),
        scratch_types=[
            pltpu.VMEM((n_pad,), jnp.float32),
            pltpu.VMEM((per_w,), jnp.int32),
            pltpu.SemaphoreType.DMA,
        ],
    )
    def hist(dst_hbm, out_hbm, hist_v, idx_v, isem):
        c = lax.axis_index("c")
        s = lax.axis_index("s")
        w = c * NS + s
        pltpu.async_copy(dst_hbm.at[pl.ds(w * per_w, per_w)], idx_v, isem)
        zero16 = jnp.zeros((16,), jnp.float32)

        @pl.loop(0, n_pad, step=16)
        def _(i):
            hist_v[pl.ds(i, 16)] = zero16

        pltpu.make_async_copy(dst_hbm.at[pl.ds(0, per_w)], idx_v, isem).wait()
        ones16 = jnp.ones((16,), jnp.float32)

        @pl.loop(0, per_w, step=16)
        def _(k):
            plsc.addupdate_scatter(hist_v, [idx_v[pl.ds(k, 16)]], ones16)

        pltpu.sync_copy(hist_v, out_hbm.at[c].at[s])

    return hist


def _edge_accumulate_sc(n_pad, d, nchunks):
    rows_per_sub = n_pad // NS
    assert nchunks % RING == 0 and nchunks >= 2 * RING

    @functools.partial(
        pl.kernel,
        out_type=jax.ShapeDtypeStruct((NC, n_pad, d), jnp.float32),
        mesh=_sc_mesh(),
        scratch_types=(
            [pltpu.VMEM_SHARED((n_pad, d), jnp.float32)]
            + [pltpu.VMEM((RING, CHUNK), jnp.int32)] * 2
            + [pltpu.VMEM((CHUNK, d), jnp.float32)] * RING
            + [pltpu.SemaphoreType.DMA] * (3 * RING + 1)
        ),
    )
    def accum(table_hbm, src_hbm, dst_hbm, zeros_hbm, out_hbm,
              acc, sidx_v, didx_v, *bufs_and_sems):
        rows = list(bufs_and_sems[:RING])
        isem = list(bufs_and_sems[RING:2 * RING])
        gsem = list(bufs_and_sems[2 * RING:3 * RING])
        ssem = list(bufs_and_sems[3 * RING:4 * RING])
        zsem = bufs_and_sems[4 * RING]
        c = lax.axis_index("c")
        s = lax.axis_index("s")
        w = c * NS + s
        row0 = s * rows_per_sub
        pltpu.async_copy(zeros_hbm.at[pl.ds(row0, rows_per_sub), :],
                        acc.at[pl.ds(row0, rows_per_sub), :], zsem)

        def issue_idx(j, b):
            pltpu.async_copy(src_hbm.at[w, j], sidx_v.at[b], isem[b])
            pltpu.async_copy(dst_hbm.at[w, j], didx_v.at[b], isem[b])

        def wait_idx(b):
            pltpu.make_async_copy(src_hbm.at[w, 0], sidx_v.at[b],
                                  isem[b]).wait()
            pltpu.make_async_copy(dst_hbm.at[w, 0], didx_v.at[b],
                                  isem[b]).wait()

        def issue_gather(b):
            pltpu.async_copy(table_hbm.at[sidx_v.at[b]], rows[b], gsem[b])

        def wait_gather(b):
            pltpu.make_async_copy(table_hbm.at[sidx_v.at[0]], rows[b],
                                  gsem[b]).wait()

        def issue_scatter(b):
            pltpu.async_copy(rows[b], acc.at[didx_v.at[b]], ssem[b], add=True)

        def wait_scatter(b):
            pltpu.make_async_copy(rows[b], acc.at[didx_v.at[0]],
                                  ssem[b]).wait()

        def stage(i, with_idx=True, with_gather=True, with_scatter=True,
                  with_free=True):
            if with_idx:
                if with_free:
                    wait_scatter(i % RING)
                issue_idx(i, i % RING)
            if with_gather:
                wait_idx((i - 1) % RING)
                issue_gather((i - 1) % RING)
            if with_scatter:
                wait_gather((i - 2) % RING)
                issue_scatter((i - 2) % RING)

        for i in range(2):
            stage(i, with_gather=i >= 1, with_scatter=False, with_free=False)
        pltpu.make_async_copy(zeros_hbm.at[pl.ds(row0, rows_per_sub), :],
                              acc.at[pl.ds(row0, rows_per_sub), :], zsem).wait()
        plsc.subcore_barrier()
        for i in range(2, RING):
            stage(i, with_gather=True, with_scatter=True, with_free=False)

        @pl.loop(1, nchunks // RING)
        def _(jq):
            for k in range(RING):
                b = k
                wait_scatter(b)
                pltpu.async_copy(src_hbm.at[w, jq * RING + k], sidx_v.at[b],
                                 isem[b])
                pltpu.async_copy(dst_hbm.at[w, jq * RING + k], didx_v.at[b],
                                 isem[b])
                wait_idx((k - 1) % RING)
                issue_gather((k - 1) % RING)
                wait_gather((k - 2) % RING)
                issue_scatter((k - 2) % RING)

        stage(nchunks, with_idx=False)
        stage(nchunks + 1, with_idx=False, with_gather=False)
        for b in range(RING):
            wait_scatter(b)
        plsc.subcore_barrier()
        pltpu.sync_copy(acc.at[pl.ds(row0, rows_per_sub), :],
                        out_hbm.at[c].at[pl.ds(row0, rows_per_sub), :])

    return accum


def _tc_matmul(x, w, rb):
    n, k = x.shape
    _, m = w.shape

    def body(x_ref, w_ref, o_ref):
        o_ref[...] = jnp.dot(x_ref[...], w_ref[...],
                             preferred_element_type=jnp.float32)

    return pl.pallas_call(
        body,
        grid=(n // rb,),
        in_specs=[pl.BlockSpec((rb, k), lambda i: (i, 0)),
                  pl.BlockSpec((k, m), lambda i: (0, 0))],
        out_specs=pl.BlockSpec((rb, m), lambda i: (i, 0)),
        out_shape=jax.ShapeDtypeStruct((n, m), jnp.float32),
    )(x, w)


def _tc_dis_prescale(hist2, x, w1, rb):
    n, k = x.shape
    d = w1.shape[1]

    def body(h_ref, x_ref, w1_ref, dis_ref, h1_ref, g_ref):
        deg = jnp.sum(h_ref[...], axis=0) + 1.0
        dis_c = lax.rsqrt(deg).reshape(rb, 1)
        dis_ref[...] = dis_c
        h1 = jnp.dot(x_ref[...], w1_ref[...], preferred_element_type=jnp.float32)
        h1_ref[...] = h1
        g_ref[...] = h1 * dis_c

    return pl.pallas_call(
        body,
        grid=(n // rb,),
        in_specs=[pl.BlockSpec((NW, rb), lambda i: (0, i)),
                  pl.BlockSpec((rb, k), lambda i: (i, 0)),
                  pl.BlockSpec((k, d), lambda i: (0, 0))],
        out_specs=[pl.BlockSpec((rb, 1), lambda i: (i, 0)),
                   pl.BlockSpec((rb, d), lambda i: (i, 0)),
                   pl.BlockSpec((rb, d), lambda i: (i, 0))],
        out_shape=[jax.ShapeDtypeStruct((n, 1), jnp.float32),
                   jax.ShapeDtypeStruct((n, d), jnp.float32),
                   jax.ShapeDtypeStruct((n, d), jnp.float32)],
    )(hist2, x, w1)


def _tc_layer_mid(acc, h1, dis, b1, w2, rb):
    _, n, d = acc.shape
    m = w2.shape[1]

    def body(acc_ref, h1_ref, dis_ref, b1_ref, w2_ref, h2_ref, g2_ref):
        dis_c = dis_ref[...]
        su = acc_ref[0] + acc_ref[1]
        z = jnp.maximum(dis_c * su + (dis_c * dis_c) * h1_ref[...] + b1_ref[...],
                        0.0)
        h2 = jnp.dot(z, w2_ref[...], preferred_element_type=jnp.float32)
        h2_ref[...] = h2
        g2_ref[...] = h2 * dis_c

    return pl.pallas_call(
        body,
        grid=(n // rb,),
        in_specs=[pl.BlockSpec((NC, rb, d), lambda i: (0, i, 0)),
                  pl.BlockSpec((rb, d), lambda i: (i, 0)),
                  pl.BlockSpec((rb, 1), lambda i: (i, 0)),
                  pl.BlockSpec((1, d), lambda i: (0, 0)),
                  pl.BlockSpec((d, m), lambda i: (0, 0))],
        out_specs=[pl.BlockSpec((rb, m), lambda i: (i, 0)),
                   pl.BlockSpec((rb, m), lambda i: (i, 0))],
        out_shape=[jax.ShapeDtypeStruct((n, m), jnp.float32),
                   jax.ShapeDtypeStruct((n, m), jnp.float32)],
    )(acc, h1, dis, b1, w2)


def _tc_layer_out(acc, h2, dis, b2, n_out, rb):
    _, _, d = acc.shape

    def body(acc_ref, h2_ref, dis_ref, b2_ref, o_ref):
        dis_c = dis_ref[...]
        su = acc_ref[0] + acc_ref[1]
        o_ref[...] = dis_c * su + (dis_c * dis_c) * h2_ref[...] + b2_ref[...]

    return pl.pallas_call(
        body,
        grid=(n_out // rb,),
        in_specs=[pl.BlockSpec((NC, rb, d), lambda i: (0, i, 0)),
                  pl.BlockSpec((rb, d), lambda i: (i, 0)),
                  pl.BlockSpec((rb, 1), lambda i: (i, 0)),
                  pl.BlockSpec((1, d), lambda i: (0, 0))],
        out_specs=pl.BlockSpec((rb, d), lambda i: (i, 0)),
        out_shape=jax.ShapeDtypeStruct((n_out, d), jnp.float32),
    )(acc, h2, dis, b2)


def kernel(x, edge_index, W1, b1, W2, b2):
    n, _ = x.shape
    e = edge_index.shape[1]
    d_h = W1.shape[1]
    d_out = W2.shape[1]

    n_pad = ((n + 80 * NS - 1) // (80 * NS)) * (80 * NS)
    rb = n_pad // 10
    assert rb % 8 == 0

    nchunks = -(-e // (NW * CHUNK))
    nchunks = max(2 * RING, ((nchunks + RING - 1) // RING) * RING)
    e_pad = NW * CHUNK * nchunks
    if e_pad > e and n_pad == n:
        n_pad += 80 * NS
    pad_len = e_pad - e
    sink = n + (jnp.arange(pad_len, dtype=jnp.int32) % (n_pad - n)) \
        if pad_len else jnp.zeros((0,), jnp.int32)
    src = jnp.concatenate([edge_index[0], sink]).reshape(NW, nchunks, CHUNK)
    dst_flat = jnp.concatenate([edge_index[1], sink])
    dst = dst_flat.reshape(NW, nchunks, CHUNK)

    x_pad = jnp.pad(x, ((0, n_pad - n), (0, 0)))
    zeros_tab = jnp.zeros((n_pad, d_h), jnp.float32)

    hist = _hist_sc(n_pad, nchunks * CHUNK)(dst_flat)
    hist2 = hist.reshape(NW, n_pad)

    dis, h1, g1 = _tc_dis_prescale(hist2, x_pad, W1, rb)

    acc1 = _edge_accumulate_sc(n_pad, d_h, nchunks)(g1, src, dst, zeros_tab)

    h2, g2 = _tc_layer_mid(acc1, h1, dis, b1.reshape(1, d_h), W2, rb)

    acc2 = _edge_accumulate_sc(n_pad, d_out, nchunks)(g2, src, dst, zeros_tab)

    rb_out = rb
    if n % rb:
        for div in (10, 8, 5, 4, 2, 1):
            if n % div == 0 and (n // div) % 8 == 0:
                rb_out = n // div
                break
    else:
        rb_out = rb if n % rb == 0 else n
    return _tc_layer_out(acc2, h2, dis, b2.reshape(1, d_out), n, rb_out)


# speedup vs baseline: 1.0567x; 1.0115x over previous
import dataclasses
import functools

import jax
import jax.numpy as jnp
from jax import lax
from jax.experimental import pallas as pl
from jax.experimental.pallas import tpu as pltpu
from jax.experimental.pallas import tpu_sc as plsc

NC, NS = 2, 16
NW = NC * NS
CHUNK = 88
RING = 4


def _sc_mesh():
    return plsc.VectorSubcoreMesh(
        core_axis_name="c", subcore_axis_name="s", num_cores=NC, num_subcores=NS
    )


def _sc_params():
    cp = pltpu.CompilerParams()
    if "needs_layout_passes" in pltpu.CompilerParams.__dataclass_fields__:
        cp = dataclasses.replace(cp, needs_layout_passes=False)
    return cp


def _hist_sc(n_pad, per_w):
    assert per_w % 16 == 0 and n_pad % 16 == 0

    @functools.partial(
        pl.kernel,
        out_type=jax.ShapeDtypeStruct((NC, NS, n_pad), jnp.float32),
        mesh=_sc_mesh(),
        compiler_params=_sc_params(),
        scratch_types=[
            pltpu.VMEM((n_pad,), jnp.float32),
            pltpu.VMEM((per_w,), jnp.int32),
            pltpu.SemaphoreType.DMA,
        ],
    )
    def hist(dst_hbm, out_hbm, hist_v, idx_v, isem):
        c = lax.axis_index("c")
        s = lax.axis_index("s")
        w = c * NS + s
        pltpu.async_copy(dst_hbm.at[pl.ds(w * per_w, per_w)], idx_v, isem)
        zero16 = jnp.zeros((16,), jnp.float32)

        @pl.loop(0, n_pad, step=16)
        def _(i):
            hist_v[pl.ds(i, 16)] = zero16

        pltpu.make_async_copy(dst_hbm.at[pl.ds(0, per_w)], idx_v, isem).wait()
        ones16 = jnp.ones((16,), jnp.float32)

        @pl.loop(0, per_w, step=16)
        def _(k):
            plsc.addupdate_scatter(hist_v, [idx_v[pl.ds(k, 16)]], ones16)

        pltpu.sync_copy(hist_v, out_hbm.at[c].at[s])

    return hist


def _edge_accumulate_sc(n_pad, d, nchunks):
    rows_per_sub = n_pad // NS
    assert nchunks % RING == 0 and nchunks >= 2 * RING

    @functools.partial(
        pl.kernel,
        out_type=jax.ShapeDtypeStruct((NC, n_pad, d), jnp.float32),
        mesh=_sc_mesh(),
        scratch_types=(
            [pltpu.VMEM_SHARED((n_pad, d), jnp.float32)]
            + [pltpu.VMEM((RING, CHUNK), jnp.int32)] * 2
            + [pltpu.VMEM((CHUNK, d), jnp.float32)] * RING
            + [pltpu.SemaphoreType.DMA] * (3 * RING + 1)
        ),
    )
    def accum(table_hbm, src_hbm, dst_hbm, zeros_hbm, out_hbm,
              acc, sidx_v, didx_v, *bufs_and_sems):
        rows = list(bufs_and_sems[:RING])
        isem = list(bufs_and_sems[RING:2 * RING])
        gsem = list(bufs_and_sems[2 * RING:3 * RING])
        ssem = list(bufs_and_sems[3 * RING:4 * RING])
        zsem = bufs_and_sems[4 * RING]
        c = lax.axis_index("c")
        s = lax.axis_index("s")
        w = c * NS + s
        row0 = s * rows_per_sub
        pltpu.async_copy(zeros_hbm.at[pl.ds(row0, rows_per_sub), :],
                        acc.at[pl.ds(row0, rows_per_sub), :], zsem)

        def issue_idx(j, b):
            pltpu.async_copy(src_hbm.at[w, j], sidx_v.at[b], isem[b])
            pltpu.async_copy(dst_hbm.at[w, j], didx_v.at[b], isem[b])

        def wait_idx(b):
            pltpu.make_async_copy(src_hbm.at[w, 0], sidx_v.at[b],
                                  isem[b]).wait()
            pltpu.make_async_copy(dst_hbm.at[w, 0], didx_v.at[b],
                                  isem[b]).wait()

        def issue_gather(b):
            pltpu.async_copy(table_hbm.at[sidx_v.at[b]], rows[b], gsem[b])

        def wait_gather(b):
            pltpu.make_async_copy(table_hbm.at[sidx_v.at[0]], rows[b],
                                  gsem[b]).wait()

        def issue_scatter(b):
            pltpu.async_copy(rows[b], acc.at[didx_v.at[b]], ssem[b], add=True)

        def wait_scatter(b):
            pltpu.make_async_copy(rows[b], acc.at[didx_v.at[0]],
                                  ssem[b]).wait()

        def stage(i, with_idx=True, with_gather=True, with_scatter=True,
                  with_free=True):
            if with_idx:
                if with_free:
                    wait_scatter(i % RING)
                issue_idx(i, i % RING)
            if with_gather:
                wait_idx((i - 1) % RING)
                issue_gather((i - 1) % RING)
            if with_scatter:
                wait_gather((i - 2) % RING)
                issue_scatter((i - 2) % RING)

        for i in range(2):
            stage(i, with_gather=i >= 1, with_scatter=False, with_free=False)
        pltpu.make_async_copy(zeros_hbm.at[pl.ds(row0, rows_per_sub), :],
                              acc.at[pl.ds(row0, rows_per_sub), :], zsem).wait()
        plsc.subcore_barrier()
        for i in range(2, RING):
            stage(i, with_gather=True, with_scatter=True, with_free=False)

        @pl.loop(1, nchunks // RING)
        def _(jq):
            for k in range(RING):
                b = k
                wait_scatter(b)
                pltpu.async_copy(src_hbm.at[w, jq * RING + k], sidx_v.at[b],
                                 isem[b])
                pltpu.async_copy(dst_hbm.at[w, jq * RING + k], didx_v.at[b],
                                 isem[b])
                wait_idx((k - 1) % RING)
                issue_gather((k - 1) % RING)
                wait_gather((k - 2) % RING)
                issue_scatter((k - 2) % RING)

        stage(nchunks, with_idx=False)
        stage(nchunks + 1, with_idx=False, with_gather=False)
        for b in range(RING):
            wait_scatter(b)
        plsc.subcore_barrier()
        pltpu.sync_copy(acc.at[pl.ds(row0, rows_per_sub), :],
                        out_hbm.at[c].at[pl.ds(row0, rows_per_sub), :])

    return accum


def _tc_matmul(x, w, rb):
    n, k = x.shape
    _, m = w.shape

    def body(x_ref, w_ref, o_ref):
        o_ref[...] = jnp.dot(x_ref[...], w_ref[...],
                             preferred_element_type=jnp.float32)

    return pl.pallas_call(
        body,
        grid=(n // rb,),
        in_specs=[pl.BlockSpec((rb, k), lambda i: (i, 0)),
                  pl.BlockSpec((k, m), lambda i: (0, 0))],
        out_specs=pl.BlockSpec((rb, m), lambda i: (i, 0)),
        out_shape=jax.ShapeDtypeStruct((n, m), jnp.float32),
    )(x, w)


def _tc_dis_prescale(hist2, x, w1, rb):
    n, k = x.shape
    d = w1.shape[1]

    def body(h_ref, x_ref, w1_ref, dis_ref, h1_ref, g_ref):
        deg = jnp.sum(h_ref[...], axis=0) + 1.0
        dis_c = lax.rsqrt(deg).reshape(rb, 1)
        dis_ref[...] = dis_c
        h1 = jnp.dot(x_ref[...], w1_ref[...], preferred_element_type=jnp.float32)
        h1_ref[...] = h1
        g_ref[...] = h1 * dis_c

    return pl.pallas_call(
        body,
        grid=(n // rb,),
        in_specs=[pl.BlockSpec((NW, rb), lambda i: (0, i)),
                  pl.BlockSpec((rb, k), lambda i: (i, 0)),
                  pl.BlockSpec((k, d), lambda i: (0, 0))],
        out_specs=[pl.BlockSpec((rb, 1), lambda i: (i, 0)),
                   pl.BlockSpec((rb, d), lambda i: (i, 0)),
                   pl.BlockSpec((rb, d), lambda i: (i, 0))],
        out_shape=[jax.ShapeDtypeStruct((n, 1), jnp.float32),
                   jax.ShapeDtypeStruct((n, d), jnp.float32),
                   jax.ShapeDtypeStruct((n, d), jnp.float32)],
    )(hist2, x, w1)


def _tc_layer_mid(acc, h1, dis, b1, w2, rb):
    _, n, d = acc.shape
    m = w2.shape[1]

    def body(acc_ref, h1_ref, dis_ref, b1_ref, w2_ref, h2_ref, g2_ref):
        dis_c = dis_ref[...]
        su = acc_ref[0] + acc_ref[1]
        z = jnp.maximum(dis_c * su + (dis_c * dis_c) * h1_ref[...] + b1_ref[...],
                        0.0)
        h2 = jnp.dot(z, w2_ref[...], preferred_element_type=jnp.float32)
        h2_ref[...] = h2
        g2_ref[...] = h2 * dis_c

    return pl.pallas_call(
        body,
        grid=(n // rb,),
        in_specs=[pl.BlockSpec((NC, rb, d), lambda i: (0, i, 0)),
                  pl.BlockSpec((rb, d), lambda i: (i, 0)),
                  pl.BlockSpec((rb, 1), lambda i: (i, 0)),
                  pl.BlockSpec((1, d), lambda i: (0, 0)),
                  pl.BlockSpec((d, m), lambda i: (0, 0))],
        out_specs=[pl.BlockSpec((rb, m), lambda i: (i, 0)),
                   pl.BlockSpec((rb, m), lambda i: (i, 0))],
        out_shape=[jax.ShapeDtypeStruct((n, m), jnp.float32),
                   jax.ShapeDtypeStruct((n, m), jnp.float32)],
    )(acc, h1, dis, b1, w2)


def _tc_layer_out(acc, h2, dis, b2, n_out, rb):
    _, _, d = acc.shape

    def body(acc_ref, h2_ref, dis_ref, b2_ref, o_ref):
        dis_c = dis_ref[...]
        su = acc_ref[0] + acc_ref[1]
        o_ref[...] = dis_c * su + (dis_c * dis_c) * h2_ref[...] + b2_ref[...]

    return pl.pallas_call(
        body,
        grid=(n_out // rb,),
        in_specs=[pl.BlockSpec((NC, rb, d), lambda i: (0, i, 0)),
                  pl.BlockSpec((rb, d), lambda i: (i, 0)),
                  pl.BlockSpec((rb, 1), lambda i: (i, 0)),
                  pl.BlockSpec((1, d), lambda i: (0, 0))],
        out_specs=pl.BlockSpec((rb, d), lambda i: (i, 0)),
        out_shape=jax.ShapeDtypeStruct((n_out, d), jnp.float32),
    )(acc, h2, dis, b2)


def kernel(x, edge_index, W1, b1, W2, b2):
    n, _ = x.shape
    e = edge_index.shape[1]
    d_h = W1.shape[1]
    d_out = W2.shape[1]

    n_pad = ((n + 80 * NS - 1) // (80 * NS)) * (80 * NS)
    rb = n_pad // 5
    assert rb % 8 == 0

    nchunks = -(-e // (NW * CHUNK))
    nchunks = max(2 * RING, ((nchunks + RING - 1) // RING) * RING)
    e_pad = NW * CHUNK * nchunks
    if e_pad > e and n_pad == n:
        n_pad += 80 * NS
    pad_len = e_pad - e
    sink = n + (jnp.arange(pad_len, dtype=jnp.int32) % (n_pad - n)) \
        if pad_len else jnp.zeros((0,), jnp.int32)
    src = jnp.concatenate([edge_index[0], sink]).reshape(NW, nchunks, CHUNK)
    dst_flat = jnp.concatenate([edge_index[1], sink])
    dst = dst_flat.reshape(NW, nchunks, CHUNK)

    x_pad = jnp.pad(x, ((0, n_pad - n), (0, 0)))
    zeros_tab = jnp.zeros((n_pad, d_h), jnp.float32)

    hist = _hist_sc(n_pad, nchunks * CHUNK)(dst_flat)
    hist2 = hist.reshape(NW, n_pad)

    dis, h1, g1 = _tc_dis_prescale(hist2, x_pad, W1, rb)

    acc1 = _edge_accumulate_sc(n_pad, d_h, nchunks)(g1, src, dst, zeros_tab)

    h2, g2 = _tc_layer_mid(acc1, h1, dis, b1.reshape(1, d_h), W2, rb)

    acc2 = _edge_accumulate_sc(n_pad, d_out, nchunks)(g2, src, dst, zeros_tab)

    rb_out = rb
    if n % rb:
        for div in (10, 8, 5, 4, 2, 1):
            if n % div == 0 and (n // div) % 8 == 0:
                rb_out = n // div
                break
    else:
        rb_out = rb if n % rb == 0 else n
    return _tc_layer_out(acc2, h2, dis, b2.reshape(1, d_out), n, rb_out)
